# baseline (speedup 1.0000x reference)
import functools

import jax
import jax.numpy as jnp
from jax import lax
from jax.experimental import pallas as pl
from jax.experimental.pallas import tpu as pltpu
from jax.experimental.pallas import tpu_sc as plsc

DIM = 64
L = 16
NC = 2
NS = 16
CHUNK = 128


def _rsqrt_vec(s):
    i = lax.bitcast_convert_type(s, jnp.int32)
    i = jnp.int32(0x5F3759DF) - (i >> 1)
    y = lax.bitcast_convert_type(i, jnp.float32)
    half = s * jnp.float32(0.5)
    for _ in range(3):
        y = y * (jnp.float32(1.5) - half * y * y)
    return y


@functools.lru_cache(maxsize=None)
def _build(batch):
    bpw = batch // (NC * NS)
    nchunk = bpw // CHUNK
    ngroup = CHUNK // L
    mesh = plsc.VectorSubcoreMesh(core_axis_name="c", subcore_axis_name="s")

    @functools.partial(
        pl.kernel,
        out_type=jax.ShapeDtypeStruct((batch // L, L), jnp.float32),
        mesh=mesh,
        compiler_params=pltpu.CompilerParams(use_tc_tiling_on_sc=False),
        scratch_types=[
            pltpu.VMEM((bpw,), jnp.int32),
            pltpu.VMEM((bpw,), jnp.int32),
            pltpu.VMEM((bpw,), jnp.int32),
            pltpu.VMEM((CHUNK, DIM), jnp.float32),
            pltpu.VMEM((CHUNK, DIM), jnp.float32),
            pltpu.VMEM((CHUNK, 2 * DIM), jnp.float32),
            pltpu.VMEM((bpw // L, L), jnp.float32),
            pltpu.SemaphoreType.DMA,
        ],
    )
    def score_kernel(h_hbm, r_hbm, t_hbm, ent_hbm, rel_hbm, out_hbm,
                     hidx, ridx, tidx, head_v, tail_v, rel_v, out_v, sem):
        wid = lax.axis_index("s") * NC + lax.axis_index("c")
        base = wid * bpw
        pltpu.sync_copy(h_hbm.at[pl.ds(base, bpw)], hidx)
        pltpu.sync_copy(r_hbm.at[pl.ds(base, bpw)], ridx)
        pltpu.sync_copy(t_hbm.at[pl.ds(base, bpw)], tidx)

        lane = lax.iota(jnp.int32, L)
        perms = [lane ^ k for k in (1, 2, 4, 8)]
        dnums = lax.GatherDimensionNumbers(
            offset_dims=(), collapsed_slice_dims=(0,), start_index_map=(0,))

        def allsum(v):
            for p in perms:
                v = v + lax.gather(
                    v, p[:, None], dnums, slice_sizes=(1,),
                    mode=lax.GatherScatterMode.PROMISE_IN_BOUNDS)
            return v

        def row_score(i):
            h0 = head_v[i, pl.ds(0 * L, L)]
            h1 = head_v[i, pl.ds(1 * L, L)]
            h2 = head_v[i, pl.ds(2 * L, L)]
            h3 = head_v[i, pl.ds(3 * L, L)]
            t0 = tail_v[i, pl.ds(0 * L, L)]
            t1 = tail_v[i, pl.ds(1 * L, L)]
            t2 = tail_v[i, pl.ds(2 * L, L)]
            t3 = tail_v[i, pl.ds(3 * L, L)]
            hs = allsum(h0 * h0 + h1 * h1 + h2 * h2 + h3 * h3)
            ts = allsum(t0 * t0 + t1 * t1 + t2 * t2 + t3 * t3)
            ih = _rsqrt_vec(jnp.maximum(hs, jnp.float32(1e-24)))
            it = _rsqrt_vec(jnp.maximum(ts, jnp.float32(1e-24)))
            r0 = rel_v[i, pl.ds(0 * L, L)]
            r1 = rel_v[i, pl.ds(1 * L, L)]
            r2 = rel_v[i, pl.ds(2 * L, L)]
            r3 = rel_v[i, pl.ds(3 * L, L)]
            r4 = rel_v[i, pl.ds(4 * L, L)]
            r5 = rel_v[i, pl.ds(5 * L, L)]
            r6 = rel_v[i, pl.ds(6 * L, L)]
            r7 = rel_v[i, pl.ds(7 * L, L)]
            acc = jnp.abs(h0 * ih * r0 - t0 * it * r4)
            acc = acc + jnp.abs(h1 * ih * r1 - t1 * it * r5)
            acc = acc + jnp.abs(h2 * ih * r2 - t2 * it * r6)
            acc = acc + jnp.abs(h3 * ih * r3 - t3 * it * r7)
            return -allsum(acc)

        for c in range(nchunk):
            off = c * CHUNK
            pltpu.async_copy(
                ent_hbm.at[hidx.at[pl.ds(off, CHUNK)]], head_v, sem).wait()
            pltpu.async_copy(
                ent_hbm.at[tidx.at[pl.ds(off, CHUNK)]], tail_v, sem).wait()
            pltpu.async_copy(
                rel_hbm.at[ridx.at[pl.ds(off, CHUNK)]], rel_v, sem).wait()

            def group(g, carry, c=c):
                pack = jnp.zeros((L,), jnp.float32)
                for kk in range(L):
                    val = row_score(g * L + kk)
                    pack = jnp.where(lane == kk, val, pack)
                out_v[c * ngroup + g, :] = pack
                return carry

            lax.fori_loop(0, ngroup, group, 0)

        pltpu.sync_copy(
            out_v, out_hbm.at[pl.ds(wid * (bpw // L), bpw // L), :])

    return score_kernel


def kernel(h, r, t, entity_emb, relation_emb):
    batch = h.shape[0]
    out = _build(batch)(h, r, t, entity_emb, relation_emb)
    return out.reshape(batch, 1)


# speedup vs baseline: 1.1141x; 1.1141x over previous
import functools

import jax
import jax.numpy as jnp
from jax import lax
from jax.experimental import pallas as pl
from jax.experimental.pallas import tpu as pltpu
from jax.experimental.pallas import tpu_sc as plsc

DIM = 64
L = 16
NC = 2
NS = 16
CHUNK = 128


def _rsqrt_vec(s):
    i = lax.bitcast_convert_type(s, jnp.int32)
    i = jnp.int32(0x5F3759DF) - (i >> 1)
    y = lax.bitcast_convert_type(i, jnp.float32)
    half = s * jnp.float32(0.5)
    for _ in range(3):
        y = y * (jnp.float32(1.5) - half * y * y)
    return y


@functools.lru_cache(maxsize=None)
def _build(batch):
    bpw = batch // (NC * NS)
    nchunk = bpw // CHUNK
    ngroup = CHUNK // L
    mesh = plsc.VectorSubcoreMesh(core_axis_name="c", subcore_axis_name="s")

    @functools.partial(
        pl.kernel,
        out_type=jax.ShapeDtypeStruct((batch // L, L), jnp.float32),
        mesh=mesh,
        compiler_params=pltpu.CompilerParams(use_tc_tiling_on_sc=False),
        scratch_types=[
            pltpu.VMEM((bpw,), jnp.int32),
            pltpu.VMEM((bpw,), jnp.int32),
            pltpu.VMEM((bpw,), jnp.int32),
            pltpu.VMEM((CHUNK, 2 * DIM), jnp.float32),
            pltpu.VMEM((CHUNK, 2 * DIM), jnp.float32),
            pltpu.VMEM((CHUNK, 2 * DIM), jnp.float32),
            pltpu.VMEM((bpw // L, L), jnp.float32),
            pltpu.SemaphoreType.DMA,
            pltpu.SemaphoreType.DMA,
            pltpu.SemaphoreType.DMA,
        ],
    )
    def score_kernel(h_hbm, r_hbm, t_hbm, ent_hbm, rel_hbm, out_hbm,
                     hidx, ridx, tidx, head_v, tail_v, rel_v, out_v,
                     sem_h, sem_t, sem_r):
        wid = lax.axis_index("s") * NC + lax.axis_index("c")
        base = wid * bpw
        pltpu.sync_copy(h_hbm.at[pl.ds(base, bpw)], hidx)
        pltpu.sync_copy(r_hbm.at[pl.ds(base, bpw)], ridx)
        pltpu.sync_copy(t_hbm.at[pl.ds(base, bpw)], tidx)

        lane = lax.iota(jnp.int32, L)
        perms = [lane ^ k for k in (1, 2, 4, 8)]
        dnums = lax.GatherDimensionNumbers(
            offset_dims=(), collapsed_slice_dims=(0,), start_index_map=(0,))

        def allsum(v):
            for p in perms:
                v = v + lax.gather(
                    v, p[:, None], dnums, slice_sizes=(1,),
                    mode=lax.GatherScatterMode.PROMISE_IN_BOUNDS)
            return v

        def row_score(i):
            h0 = head_v[i, pl.ds(0 * L, L)]
            h1 = head_v[i, pl.ds(1 * L, L)]
            h2 = head_v[i, pl.ds(2 * L, L)]
            h3 = head_v[i, pl.ds(3 * L, L)]
            t0 = tail_v[i, pl.ds(0 * L, L)]
            t1 = tail_v[i, pl.ds(1 * L, L)]
            t2 = tail_v[i, pl.ds(2 * L, L)]
            t3 = tail_v[i, pl.ds(3 * L, L)]
            hs = allsum(h0 * h0 + h1 * h1 + h2 * h2 + h3 * h3)
            ts = allsum(t0 * t0 + t1 * t1 + t2 * t2 + t3 * t3)
            ih = _rsqrt_vec(jnp.maximum(hs, jnp.float32(1e-24)))
            it = _rsqrt_vec(jnp.maximum(ts, jnp.float32(1e-24)))
            r0 = rel_v[i, pl.ds(0 * L, L)]
            r1 = rel_v[i, pl.ds(1 * L, L)]
            r2 = rel_v[i, pl.ds(2 * L, L)]
            r3 = rel_v[i, pl.ds(3 * L, L)]
            r4 = rel_v[i, pl.ds(4 * L, L)]
            r5 = rel_v[i, pl.ds(5 * L, L)]
            r6 = rel_v[i, pl.ds(6 * L, L)]
            r7 = rel_v[i, pl.ds(7 * L, L)]
            acc = jnp.abs(h0 * ih * r0 - t0 * it * r4)
            acc = acc + jnp.abs(h1 * ih * r1 - t1 * it * r5)
            acc = acc + jnp.abs(h2 * ih * r2 - t2 * it * r6)
            acc = acc + jnp.abs(h3 * ih * r3 - t3 * it * r7)
            return -allsum(acc)

        for c in range(nchunk):
            off = c * CHUNK
            cp_h = pltpu.async_copy(
                ent_hbm.at[hidx.at[pl.ds(off, CHUNK)]], head_v, sem_h)
            cp_t = pltpu.async_copy(
                ent_hbm.at[tidx.at[pl.ds(off, CHUNK)]], tail_v, sem_t)
            cp_r = pltpu.async_copy(
                rel_hbm.at[ridx.at[pl.ds(off, CHUNK)]], rel_v, sem_r)
            cp_h.wait()
            cp_t.wait()
            cp_r.wait()

            def group(g, carry, c=c):
                pack = jnp.zeros((L,), jnp.float32)
                for kk in range(L):
                    val = row_score(g * L + kk)
                    pack = jnp.where(lane == kk, val, pack)
                out_v[c * ngroup + g, :] = pack
                return carry

            lax.fori_loop(0, ngroup, group, 0)

        pltpu.sync_copy(
            out_v, out_hbm.at[pl.ds(wid * (bpw // L), bpw // L), :])

    return score_kernel


def kernel(h, r, t, entity_emb, relation_emb):
    batch = h.shape[0]
    ep = jnp.pad(entity_emb, ((0, 0), (0, DIM)))
    out = _build(batch)(h, r, t, ep, relation_emb)
    return out.reshape(batch, 1)


# speedup vs baseline: 1.3837x; 1.2420x over previous
import functools

import jax
import jax.numpy as jnp
from jax import lax
from jax.experimental import pallas as pl
from jax.experimental.pallas import tpu as pltpu
from jax.experimental.pallas import tpu_sc as plsc

DIM = 64
L = 16
NC = 2
NS = 16
NW = NC * NS
TILE = 128
WTILES = 4
WIN = WTILES * TILE
RCHUNK = 128


def _cdiv(a, b):
    return (a + b - 1) // b


@functools.lru_cache(maxsize=None)
def _build_sc(batch, n_entity):
    bpw = batch // NW
    ntiles = _cdiv(n_entity, TILE)
    full_tiles = n_entity // TILE
    tail_base = full_tiles * TILE
    tail_n = n_entity - tail_base
    npieces = _cdiv(batch, 2048)
    arena_cap = 2 * batch + L
    mesh = plsc.VectorSubcoreMesh(core_axis_name="c", subcore_axis_name="s")

    out_types = (
        jax.ShapeDtypeStruct((batch + NW, 2 * DIM), jnp.float32),
        jax.ShapeDtypeStruct((batch + NW, 2 * DIM), jnp.float32),
        jax.ShapeDtypeStruct((batch, 2 * DIM), jnp.float32),
    )

    @functools.partial(
        pl.kernel,
        out_type=out_types,
        mesh=mesh,
        compiler_params=pltpu.CompilerParams(needs_layout_passes=False),
        scratch_types=[
            pltpu.VMEM((DIM, WIN), jnp.float32),
            pltpu.VMEM((arena_cap,), jnp.int32),
            pltpu.VMEM((arena_cap,), jnp.int32),
            pltpu.VMEM((2048,), jnp.int32),
            pltpu.VMEM((L, 2 * DIM), jnp.float32),
            pltpu.VMEM((L, 2 * DIM), jnp.float32),
            pltpu.VMEM((1, L), jnp.int32),
            pltpu.VMEM((1, L), jnp.int32),
            pltpu.VMEM((RCHUNK, 2 * DIM), jnp.float32),
            pltpu.VMEM((bpw,), jnp.int32),
            pltpu.SemaphoreType.DMA,
            pltpu.SemaphoreType.DMA,
            pltpu.SemaphoreType.DMA,
            pltpu.SemaphoreType.DMA,
        ],
    )
    def sc_kernel(h_hbm, r_hbm, t_hbm, et_hbm, rel_hbm, tail_hbm,
                  u_hbm, v_hbm, rout_hbm,
                  win, ak, ap, piece, stg_u, stg_v, six_u, six_v,
                  rel_v, ridx, sem_w, sem_u, sem_v, sem_r):
        arena_cap = 2 * batch + L
        wid = lax.axis_index("s") * NC + lax.axis_index("c")
        base = wid * bpw
        lane = lax.iota(jnp.int32, L)
        perms = [lane ^ k for k in (1, 2, 4, 8)]
        dnums = lax.GatherDimensionNumbers(
            offset_dims=(), collapsed_slice_dims=(0,), start_index_map=(0,))

        def lperm(v, p):
            return lax.gather(v, p[:, None], dnums, slice_sizes=(1,),
                              mode=lax.GatherScatterMode.PROMISE_IN_BOUNDS)

        pltpu.sync_copy(r_hbm.at[pl.ds(base, bpw)], ridx)
        for cc in range(bpw // RCHUNK):
            pltpu.async_copy(
                rel_hbm.at[ridx.at[pl.ds(cc * RCHUNK, RCHUNK)]], rel_v,
                sem_r).wait()
            pltpu.sync_copy(
                rel_v, rout_hbm.at[pl.ds(base + cc * RCHUNK, RCHUNK), :])

        t0 = (wid * ntiles) // NW
        t1 = ((wid + 1) * ntiles) // NW
        t1n = jnp.minimum(t1, jnp.int32(full_tiles))

        def scan_piece(p, ptr, src_hbm, tbl_bit):
            pltpu.sync_copy(src_hbm.at[pl.ds(p * 2048, 2048)], piece)

            def vstep(i, ptr):
                v = piece[pl.ds(pl.multiple_of(i * L, L), L)]
                tl = v >> 7
                m = (tl >= t0) & (tl < t1)
                pc = jnp.where(m, jnp.int32(1), jnp.int32(0))
                for k in (1, 2, 4, 8):
                    pc = pc + jnp.where(lane >= k, lperm(pc, lane - k), 0)
                dest = jnp.where(m, ptr + pc - 1,
                                 jnp.int32(arena_cap - L))
                pos = jnp.int32(p * 2048) + i * L + lane + tbl_bit
                plsc.store_scatter(ak, [dest], v)
                plsc.store_scatter(ap, [dest], pos)
                return ptr + lperm(pc, jnp.broadcast_to(jnp.int32(L - 1),
                                                        (L,)))

            return lax.fori_loop(0, 2048 // L, vstep, ptr)

        ptr = jnp.zeros((L,), jnp.int32)
        for p in range(npieces):
            ptr = scan_piece(p, ptr, h_hbm, jnp.int32(0))
        for p in range(npieces):
            ptr = scan_piece(p, ptr, t_hbm, jnp.int32(1 << 14))
        nhits = ptr[0]
        navr = (nhits + (L - 1)) >> 4

        has_tail = jnp.where(t1 > t1n, jnp.int32(1), jnp.int32(0))
        nwin = (t1n - t0 + (WTILES - 1)) // WTILES + has_tail
        trash = jnp.broadcast_to(jnp.int32(batch), (L,)) + wid

        def window(w, carry):
            su, sv, pu, pv = carry
            is_tail = (w == nwin - 1) & (has_tail == 1)
            tc = jnp.minimum(t0 + w * WTILES, t1n - WTILES)

            @pl.when(jnp.logical_not(is_tail))
            def _():
                cb = pl.multiple_of(tc * TILE, TILE)
                pltpu.async_copy(
                    et_hbm.at[:, pl.ds(cb, WIN)], win, sem_w).wait()

            @pl.when(is_tail)
            def _():
                pltpu.async_copy(
                    tail_hbm, win.at[:, pl.ds(0, TILE)], sem_w).wait()

            lo = jnp.where(is_tail, jnp.int32(full_tiles), t0 + w * WTILES)
            hi = jnp.where(is_tail, jnp.int32(ntiles),
                           jnp.minimum(t0 + w * WTILES + WTILES, t1n))
            colbase = jnp.where(is_tail, jnp.int32(tail_base), tc * TILE)

            def avreg(g, carry):
                su, sv, pu, pv = carry
                kv = ak[pl.ds(pl.multiple_of(g * L, L), L)]
                pvv = ap[pl.ds(pl.multiple_of(g * L, L), L)]
                valid = lane < (nhits - g * L)
                tl = kv >> 7
                m = valid & (tl >= lo) & (tl < hi)

                def hit_cond(st):
                    return jnp.any(st[0])

                def hit_body(st):
                    m, su, sv, pu, pv = st
                    mn = jnp.where(m, lane, jnp.int32(L))
                    for pp in perms:
                        mn = jnp.minimum(mn, lperm(mn, pp))
                    mn = jnp.minimum(mn, jnp.int32(L - 1))
                    e_v = lperm(kv, mn)
                    p_v = lperm(pvv, mn)
                    b = p_v[0] & jnp.int32((1 << 14) - 1)
                    is_u = (p_v[0] >> 14) == 0
                    col = jnp.broadcast_to(e_v[0], (L,)) - colbase
                    rows = [
                        plsc.load_gather(
                            win, [lane + jnp.int32(k * L), col])
                        for k in range(DIM // L)
                    ]

                    @pl.when(is_u)
                    def _():
                        for k in range(DIM // L):
                            stg_u[su, pl.ds(k * L, L)] = rows[k]

                    @pl.when(jnp.logical_not(is_u))
                    def _():
                        for k in range(DIM // L):
                            stg_v[sv, pl.ds(k * L, L)] = rows[k]

                    bs = jnp.broadcast_to(b, (L,))
                    pu = jnp.where(is_u & (lane == su), bs, pu)
                    pv2 = jnp.where((~is_u) & (lane == sv), bs, pv)

                    @pl.when(is_u & (su == L - 1))
                    def _(pu=pu):
                        six_u[0, pl.ds(0, L)] = pu
                        pltpu.async_copy(
                            stg_u, u_hbm.at[six_u.at[0]], sem_u).wait()

                    @pl.when((~is_u) & (sv == L - 1))
                    def _(pv2=pv2):
                        six_v[0, pl.ds(0, L)] = pv2
                        pltpu.async_copy(
                            stg_v, v_hbm.at[six_v.at[0]], sem_v).wait()

                    su2 = jnp.where(is_u, (su + 1) & (L - 1), su)
                    sv2 = jnp.where(is_u, sv, (sv + 1) & (L - 1))
                    pu2 = jnp.where(is_u & (su == L - 1), trash, pu)
                    pv3 = jnp.where((~is_u) & (sv == L - 1), trash, pv2)
                    m2 = m & (lane != mn)
                    return (m2, su2, sv2, pu2, pv3)

                st = lax.while_loop(hit_cond, hit_body,
                                    (m, su, sv, pu, pv))
                return st[1:]

            return lax.fori_loop(0, navr, avreg, (su, sv, pu, pv))

        init = (jnp.int32(0), jnp.int32(0), trash, trash)
        su, sv, pu, pv = lax.fori_loop(0, nwin, window, init)

        six_u[0, pl.ds(0, L)] = pu
        pltpu.async_copy(stg_u, u_hbm.at[six_u.at[0]], sem_u).wait()
        six_v[0, pl.ds(0, L)] = pv
        pltpu.async_copy(stg_v, v_hbm.at[six_v.at[0]], sem_v).wait()

    return sc_kernel


def _tc_score(u_ref, v_ref, r_ref, o_ref):
    u = u_ref[:, :DIM]
    v = v_ref[:, :DIM]
    rh = r_ref[:, :DIM]
    rt = r_ref[:, DIM:]
    hn = jnp.sqrt(jnp.sum(u * u, axis=1, keepdims=True))
    tn = jnp.sqrt(jnp.sum(v * v, axis=1, keepdims=True))
    un = u / jnp.maximum(hn, 1e-12)
    vn = v / jnp.maximum(tn, 1e-12)
    o_ref[...] = -jnp.sum(jnp.abs(un * rh - vn * rt), axis=1, keepdims=True)


@functools.lru_cache(maxsize=None)
def _build_tc(batch):
    blk = 512
    return pl.pallas_call(
        _tc_score,
        grid=(batch // blk,),
        in_specs=[
            pl.BlockSpec((blk, 2 * DIM), lambda i: (i, 0)),
            pl.BlockSpec((blk, 2 * DIM), lambda i: (i, 0)),
            pl.BlockSpec((blk, 2 * DIM), lambda i: (i, 0)),
        ],
        out_specs=pl.BlockSpec((blk, 1), lambda i: (i, 0)),
        out_shape=jax.ShapeDtypeStruct((batch, 1), jnp.float32),
    )


def kernel(h, r, t, entity_emb, relation_emb):
    batch = h.shape[0]
    n_entity = entity_emb.shape[0]
    tail_base = (n_entity // TILE) * TILE
    et = entity_emb.T
    tail = lax.slice(entity_emb, (tail_base, 0), (n_entity, DIM)).T
    tail = jnp.pad(tail, ((0, 0), (0, TILE - tail.shape[1])))
    u, v, ro = _build_sc(batch, n_entity)(h, r, t, et, relation_emb, tail)
    return _build_tc(batch)(u[:batch], v[:batch], ro)


# speedup vs baseline: 1.3939x; 1.0074x over previous
import functools

import jax
import jax.numpy as jnp
from jax import lax
from jax.experimental import pallas as pl
from jax.experimental.pallas import tpu as pltpu
from jax.experimental.pallas import tpu_sc as plsc

DIM = 64
L = 16
NC = 2
NS = 16
NW = NC * NS
TILE = 128
WTILES = 4
WIN = WTILES * TILE
RCHUNK = 128


def _cdiv(a, b):
    return (a + b - 1) // b


@functools.lru_cache(maxsize=None)
def _build_sc(batch, n_entity):
    bpw = batch // NW
    ntiles = _cdiv(n_entity, TILE)
    full_tiles = n_entity // TILE
    tail_base = full_tiles * TILE
    npieces = _cdiv(batch, 2048)
    rcap = 2 * (batch // L) + 8
    mesh = plsc.VectorSubcoreMesh(core_axis_name="c", subcore_axis_name="s")

    out_types = (
        jax.ShapeDtypeStruct((batch + NW, 2 * DIM), jnp.float32),
        jax.ShapeDtypeStruct((batch + NW, 2 * DIM), jnp.float32),
        jax.ShapeDtypeStruct((batch, 2 * DIM), jnp.float32),
    )

    @functools.partial(
        pl.kernel,
        out_type=out_types,
        mesh=mesh,
        compiler_params=pltpu.CompilerParams(needs_layout_passes=False),
        scratch_types=[
            pltpu.VMEM((DIM, WIN), jnp.float32),
            pltpu.VMEM((DIM, WIN), jnp.float32),
            pltpu.VMEM((L, rcap), jnp.int32),
            pltpu.VMEM((2048,), jnp.int32),
            pltpu.VMEM((L, 2 * DIM), jnp.float32),
            pltpu.VMEM((L, 2 * DIM), jnp.float32),
            pltpu.VMEM((1, L), jnp.int32),
            pltpu.VMEM((1, L), jnp.int32),
            pltpu.VMEM((RCHUNK, 2 * DIM), jnp.float32),
            pltpu.VMEM((bpw,), jnp.int32),
            pltpu.SemaphoreType.DMA,
            pltpu.SemaphoreType.DMA,
            pltpu.SemaphoreType.DMA,
            pltpu.SemaphoreType.DMA,
            pltpu.SemaphoreType.DMA,
        ],
    )
    def sc_kernel(h_hbm, r_hbm, t_hbm, et_hbm, rel_hbm, tail_hbm,
                  u_hbm, v_hbm, rout_hbm,
                  win0, win1, aren, piece, stg_u, stg_v, six_u, six_v,
                  rel_v, ridx, sem_w0, sem_w1, sem_u, sem_v, sem_r):
        wid = lax.axis_index("s") * NC + lax.axis_index("c")
        base = wid * bpw
        lane = lax.iota(jnp.int32, L)
        perms = [lane ^ k for k in (1, 2, 4, 8)]
        dnums = lax.GatherDimensionNumbers(
            offset_dims=(), collapsed_slice_dims=(0,), start_index_map=(0,))

        def lperm(v, p):
            return lax.gather(v, p[:, None], dnums, slice_sizes=(1,),
                              mode=lax.GatherScatterMode.PROMISE_IN_BOUNDS)

        pltpu.sync_copy(r_hbm.at[pl.ds(base, bpw)], ridx)
        for cc in range(bpw // RCHUNK):
            pltpu.async_copy(
                rel_hbm.at[ridx.at[pl.ds(cc * RCHUNK, RCHUNK)]], rel_v,
                sem_r).wait()
            pltpu.sync_copy(
                rel_v, rout_hbm.at[pl.ds(base + cc * RCHUNK, RCHUNK), :])

        t0 = (wid * ntiles) // NW
        t1 = ((wid + 1) * ntiles) // NW
        t1n = jnp.minimum(t1, jnp.int32(full_tiles))
        has_tail = jnp.where(t1 > t1n, jnp.int32(1), jnp.int32(0))
        nwin = (t1n - t0 + (WTILES - 1)) // WTILES + has_tail

        def scan_piece(p, cnt, src_hbm, tbl_bit):
            pltpu.sync_copy(src_hbm.at[pl.ds(p * 2048, 2048)], piece)

            def vstep(i, cnt):
                v = piece[pl.ds(pl.multiple_of(i * L, L), L)]
                tl = v >> 7
                m = (tl >= t0) & (tl < t1)
                is_tl = tl >= jnp.int32(full_tiles)
                wv = jnp.where(is_tl, nwin - 1, (tl - t0) >> 2)
                tcv = jnp.minimum(t0 + ((tl - t0) >> 2) * WTILES,
                                  t1n - WTILES)
                colv = jnp.where(is_tl, v - jnp.int32(tail_base),
                                 v - tcv * TILE)
                pos = jnp.int32(p * 2048) + i * L + lane
                entry = (wv << 24) | (colv << 15) | tbl_bit | pos
                dest = jnp.where(m, cnt, jnp.int32(rcap - 8))
                plsc.store_scatter(aren, [lane, dest], entry)
                return cnt + jnp.where(m, 1, 0)

            return lax.fori_loop(0, 2048 // L, vstep, cnt)

        cnt = jnp.zeros((L,), jnp.int32)
        for p in range(npieces):
            cnt = scan_piece(p, cnt, h_hbm, jnp.int32(0))
        for p in range(npieces):
            cnt = scan_piece(p, cnt, t_hbm, jnp.int32(1 << 14))

        trash = jnp.broadcast_to(jnp.int32(batch), (L,)) + wid
        wins = (win0, win1)
        wsems = (sem_w0, sem_w1)

        def issue(w, buf, sem):
            is_tail = (w == nwin - 1) & (has_tail == 1)

            @pl.when(jnp.logical_not(is_tail))
            def _():
                tc = jnp.minimum(t0 + w * WTILES, t1n - WTILES)
                cb = pl.multiple_of(tc * TILE, TILE)
                pltpu.async_copy(et_hbm.at[:, pl.ds(cb, WIN)], buf, sem)

            @pl.when(is_tail)
            def _():
                pltpu.async_copy(tail_hbm, buf.at[:, pl.ds(0, TILE)], sem)

        def drain(w, buf, sem):
            is_tail = (w == nwin - 1) & (has_tail == 1)

            @pl.when(jnp.logical_not(is_tail))
            def _():
                pltpu.make_async_copy(
                    et_hbm.at[:, pl.ds(0, WIN)], buf, sem).wait()

            @pl.when(is_tail)
            def _():
                pltpu.make_async_copy(
                    tail_hbm, buf.at[:, pl.ds(0, TILE)], sem).wait()

        issue(jnp.int32(0), win0, sem_w0)

        @pl.when(nwin > 1)
        def _():
            issue(jnp.int32(1), win1, sem_w1)

        cjs = [lperm(cnt, jnp.broadcast_to(jnp.int32(j), (L,)))[0]
               for j in range(L)]

        def walk(w, buf, carry):
            su, sv, pu, pv = carry

            for j in range(L):
                cj = cjs[j]

                def avreg(g, carry, j=j):
                    su, sv, pu, pv = carry
                    ev = aren[j, pl.ds(pl.multiple_of(g * L, L), L)]
                    valid = (g * L + lane) < cjs[j]
                    m = valid & ((ev >> 24) == w)

                    def hit_cond(st):
                        return jnp.any(st[0])

                    def hit_body(st):
                        m, su, sv, pu, pv = st
                        mn = jnp.where(m, lane, jnp.int32(L))
                        for pp in perms:
                            mn = jnp.minimum(mn, lperm(mn, pp))
                        mn = jnp.minimum(mn, jnp.int32(L - 1))
                        p_v = lperm(ev, mn)
                        ent = p_v[0]
                        b = ent & jnp.int32((1 << 14) - 1)
                        is_u = ((ent >> 14) & 1) == 0
                        col = jnp.broadcast_to((ent >> 15), (L,)) & 511
                        rows = [
                            plsc.load_gather(
                                buf, [lane + jnp.int32(k * L), col])
                            for k in range(DIM // L)
                        ]

                        @pl.when(is_u)
                        def _():
                            for k in range(DIM // L):
                                stg_u[su, pl.ds(k * L, L)] = rows[k]

                        @pl.when(jnp.logical_not(is_u))
                        def _():
                            for k in range(DIM // L):
                                stg_v[sv, pl.ds(k * L, L)] = rows[k]

                        bs = jnp.broadcast_to(b, (L,))
                        pu = jnp.where(is_u & (lane == su), bs, pu)
                        pv2 = jnp.where((~is_u) & (lane == sv), bs, pv)

                        @pl.when(is_u & (su == L - 1))
                        def _(pu=pu):
                            six_u[0, pl.ds(0, L)] = pu
                            pltpu.async_copy(
                                stg_u, u_hbm.at[six_u.at[0]], sem_u).wait()

                        @pl.when((~is_u) & (sv == L - 1))
                        def _(pv2=pv2):
                            six_v[0, pl.ds(0, L)] = pv2
                            pltpu.async_copy(
                                stg_v, v_hbm.at[six_v.at[0]], sem_v).wait()

                        su2 = jnp.where(is_u, (su + 1) & (L - 1), su)
                        sv2 = jnp.where(is_u, sv, (sv + 1) & (L - 1))
                        pu2 = jnp.where(is_u & (su == L - 1), trash, pu)
                        pv3 = jnp.where((~is_u) & (sv == L - 1), trash,
                                        pv2)
                        m2 = m & (lane != mn)
                        return (m2, su2, sv2, pu2, pv3)

                    st = lax.while_loop(hit_cond, hit_body,
                                        (m, su, sv, pu, pv))
                    return st[1:]

                nv = (cj + (L - 1)) >> 4
                su, sv, pu, pv = lax.fori_loop(0, nv, avreg,
                                               (su, sv, pu, pv))
            return (su, sv, pu, pv)

        def wpair(wp, carry):
            for par in range(2):
                w = wp * 2 + par
                buf, sem = wins[par], wsems[par]

                def step(carry=carry, w=w, buf=buf, sem=sem):
                    drain(w, buf, sem)
                    carry = walk(w, buf, carry)

                    @pl.when(w + 2 < nwin)
                    def _():
                        issue(w + 2, buf, sem)

                    return carry

                carry = lax.cond(w < nwin, step, lambda c=carry: c)
            return carry

        init = (jnp.int32(0), jnp.int32(0), trash, trash)
        nwp = (nwin + 1) >> 1
        su, sv, pu, pv = lax.fori_loop(0, nwp, wpair, init)

        six_u[0, pl.ds(0, L)] = pu
        pltpu.async_copy(stg_u, u_hbm.at[six_u.at[0]], sem_u).wait()
        six_v[0, pl.ds(0, L)] = pv
        pltpu.async_copy(stg_v, v_hbm.at[six_v.at[0]], sem_v).wait()

    return sc_kernel


def _tc_score(u_ref, v_ref, r_ref, o_ref):
    u = u_ref[:, :DIM]
    v = v_ref[:, :DIM]
    rh = r_ref[:, :DIM]
    rt = r_ref[:, DIM:]
    hn = jnp.sqrt(jnp.sum(u * u, axis=1, keepdims=True))
    tn = jnp.sqrt(jnp.sum(v * v, axis=1, keepdims=True))
    un = u / jnp.maximum(hn, 1e-12)
    vn = v / jnp.maximum(tn, 1e-12)
    o_ref[...] = -jnp.sum(jnp.abs(un * rh - vn * rt), axis=1, keepdims=True)


@functools.lru_cache(maxsize=None)
def _build_tc(batch):
    blk = 512
    return pl.pallas_call(
        _tc_score,
        grid=(batch // blk,),
        in_specs=[
            pl.BlockSpec((blk, 2 * DIM), lambda i: (i, 0)),
            pl.BlockSpec((blk, 2 * DIM), lambda i: (i, 0)),
            pl.BlockSpec((blk, 2 * DIM), lambda i: (i, 0)),
        ],
        out_specs=pl.BlockSpec((blk, 1), lambda i: (i, 0)),
        out_shape=jax.ShapeDtypeStruct((batch, 1), jnp.float32),
    )


def kernel(h, r, t, entity_emb, relation_emb):
    batch = h.shape[0]
    n_entity = entity_emb.shape[0]
    tail_base = (n_entity // TILE) * TILE
    et = entity_emb.T
    tail = lax.slice(entity_emb, (tail_base, 0), (n_entity, DIM)).T
    tail = jnp.pad(tail, ((0, 0), (0, TILE - tail.shape[1])))
    u, v, ro = _build_sc(batch, n_entity)(h, r, t, et, relation_emb, tail)
    return _build_tc(batch)(u[:batch], v[:batch], ro)


# speedup vs baseline: 1.6592x; 1.1903x over previous
import functools

import jax
import jax.numpy as jnp
from jax import lax
from jax.experimental import pallas as pl
from jax.experimental.pallas import tpu as pltpu
from jax.experimental.pallas import tpu_sc as plsc

DIM = 64
L = 16
NC = 2
NS = 16
NW = NC * NS
TILE = 128
WTILES = 4
WIN = WTILES * TILE
RCHUNK = 128


def _cdiv(a, b):
    return (a + b - 1) // b


@functools.lru_cache(maxsize=None)
def _build_sc(batch, n_entity):
    bpw = batch // NW
    ntiles = _cdiv(n_entity, TILE)
    full_tiles = n_entity // TILE
    tail_base = full_tiles * TILE
    npieces = _cdiv(batch, 2048)
    rcap = 2 * (batch // L) + L
    mesh = plsc.VectorSubcoreMesh(core_axis_name="c", subcore_axis_name="s")

    out_types = (
        jax.ShapeDtypeStruct((2 * batch + NW, 2 * DIM), jnp.float32),
        jax.ShapeDtypeStruct((batch, 2 * DIM), jnp.float32),
    )

    @functools.partial(
        pl.kernel,
        out_type=out_types,
        mesh=mesh,
        compiler_params=pltpu.CompilerParams(needs_layout_passes=False),
        scratch_types=[
            pltpu.VMEM((DIM, WIN), jnp.float32),
            pltpu.VMEM((DIM, WIN), jnp.float32),
            pltpu.VMEM((L, rcap), jnp.int32),
            pltpu.VMEM((2048,), jnp.int32),
            pltpu.VMEM((L, 2 * DIM), jnp.float32),
            pltpu.VMEM((1, L), jnp.int32),
            pltpu.VMEM((RCHUNK, 2 * DIM), jnp.float32),
            pltpu.VMEM((bpw,), jnp.int32),
            pltpu.SemaphoreType.DMA,
            pltpu.SemaphoreType.DMA,
            pltpu.SemaphoreType.DMA,
            pltpu.SemaphoreType.DMA,
        ],
    )
    def sc_kernel(h_hbm, r_hbm, t_hbm, et_hbm, rel_hbm, tail_hbm,
                  uv_hbm, rout_hbm,
                  win0, win1, aren, piece, stg, six,
                  rel_v, ridx, sem_w0, sem_w1, sem_s, sem_r):
        wid = lax.axis_index("s") * NC + lax.axis_index("c")
        base = wid * bpw
        lane = lax.iota(jnp.int32, L)
        perms = [lane ^ k for k in (1, 2, 4, 8)]
        dnums = lax.GatherDimensionNumbers(
            offset_dims=(), collapsed_slice_dims=(0,), start_index_map=(0,))

        def lperm(v, p):
            return lax.gather(v, p[:, None], dnums, slice_sizes=(1,),
                              mode=lax.GatherScatterMode.PROMISE_IN_BOUNDS)

        pltpu.sync_copy(r_hbm.at[pl.ds(base, bpw)], ridx)
        for cc in range(bpw // RCHUNK):
            pltpu.async_copy(
                rel_hbm.at[ridx.at[pl.ds(cc * RCHUNK, RCHUNK)]], rel_v,
                sem_r).wait()
            pltpu.sync_copy(
                rel_v, rout_hbm.at[pl.ds(base + cc * RCHUNK, RCHUNK), :])

        t0 = (wid * ntiles) // NW
        t1 = ((wid + 1) * ntiles) // NW
        t1n = jnp.minimum(t1, jnp.int32(full_tiles))
        has_tail = jnp.where(t1 > t1n, jnp.int32(1), jnp.int32(0))
        nwin = (t1n - t0 + (WTILES - 1)) // WTILES + has_tail

        def scan_piece(p, cnt, src_hbm, tbl_bit):
            pltpu.sync_copy(src_hbm.at[pl.ds(p * 2048, 2048)], piece)

            def vstep(i, cnt):
                v = piece[pl.ds(pl.multiple_of(i * L, L), L)]
                tl = v >> 7
                m = (tl >= t0) & (tl < t1)
                is_tl = tl >= jnp.int32(full_tiles)
                wv = jnp.where(is_tl, nwin - 1, (tl - t0) >> 2)
                tcv = jnp.minimum(t0 + ((tl - t0) >> 2) * WTILES,
                                  t1n - WTILES)
                colv = jnp.where(is_tl, v - jnp.int32(tail_base),
                                 v - tcv * TILE)
                pos = jnp.int32(p * 2048) + i * L + lane
                entry = (wv << 24) | (colv << 15) | tbl_bit | pos
                entry = jnp.where(m, entry, jnp.int32(63 << 24))
                dest = jnp.where(m, cnt, jnp.int32(rcap - L))
                plsc.store_scatter(aren, [lane, dest], entry)
                return cnt + jnp.where(m, 1, 0)

            return lax.fori_loop(0, 2048 // L, vstep, cnt)

        sentinel = jnp.broadcast_to(jnp.int32(63 << 24), (L,))

        def ainit(g, c):
            for j in range(L):
                aren[j, pl.ds(pl.multiple_of(g * L, L), L)] = sentinel
            return c

        lax.fori_loop(0, rcap // L, ainit, 0)

        cnt = jnp.zeros((L,), jnp.int32)
        for p in range(npieces):
            cnt = scan_piece(p, cnt, h_hbm, jnp.int32(0))
        for p in range(npieces):
            cnt = scan_piece(p, cnt, t_hbm, jnp.int32(1 << 14))

        trash = jnp.broadcast_to(jnp.int32(2 * batch), (L,)) + wid
        wins = (win0, win1)
        wsems = (sem_w0, sem_w1)

        def issue(w, buf, sem):
            is_tail = (w == nwin - 1) & (has_tail == 1)

            @pl.when(jnp.logical_not(is_tail))
            def _():
                tc = jnp.minimum(t0 + w * WTILES, t1n - WTILES)
                cb = pl.multiple_of(tc * TILE, TILE)
                pltpu.async_copy(et_hbm.at[:, pl.ds(cb, WIN)], buf, sem)

            @pl.when(is_tail)
            def _():
                pltpu.async_copy(tail_hbm, buf.at[:, pl.ds(0, TILE)], sem)

        def drain(w, buf, sem):
            is_tail = (w == nwin - 1) & (has_tail == 1)

            @pl.when(jnp.logical_not(is_tail))
            def _():
                pltpu.make_async_copy(
                    et_hbm.at[:, pl.ds(0, WIN)], buf, sem).wait()

            @pl.when(is_tail)
            def _():
                pltpu.make_async_copy(
                    tail_hbm, buf.at[:, pl.ds(0, TILE)], sem).wait()

        issue(jnp.int32(0), win0, sem_w0)

        @pl.when(nwin > 1)
        def _():
            issue(jnp.int32(1), win1, sem_w1)

        cmax = cnt
        for pp in perms:
            cmax = jnp.maximum(cmax, lperm(cmax, pp))
        nvmax = (cmax[0] + (L - 1)) >> 4

        def walk(w, buf, carry):

            def avreg(g, carry):
                goff = pl.multiple_of(g * L, L)
                evs = [aren[j, pl.ds(goff, L)] for j in range(L)]
                ms = [(ev >> 24) == w for ev in evs]

                def hit_cond(st):
                    return jnp.any(st[0])

                def hit_body(st, ev=None):
                    m, su, pu = st
                    mn = jnp.where(m, lane, jnp.int32(L))
                    for pp in perms:
                        mn = jnp.minimum(mn, lperm(mn, pp))
                    mn = jnp.minimum(mn, jnp.int32(L - 1))
                    p_v = lperm(ev, mn)
                    ent = p_v[0]
                    b = ent & jnp.int32((1 << 15) - 1)
                    col = jnp.broadcast_to((ent >> 15), (L,)) & 511
                    for k in range(DIM // L):
                        stg[su, pl.ds(k * L, L)] = plsc.load_gather(
                            buf, [lane + jnp.int32(k * L), col])
                    pu = jnp.where(lane == su, jnp.broadcast_to(b, (L,)),
                                   pu)

                    @pl.when(su == L - 1)
                    def _(pu=pu):
                        six[0, pl.ds(0, L)] = pu
                        pltpu.async_copy(
                            stg, uv_hbm.at[six.at[0]], sem_s).wait()

                    su2 = (su + 1) & (L - 1)
                    pu2 = jnp.where(su == L - 1, trash, pu)
                    m2 = m & (lane != mn)
                    return (m2, su2, pu2)

                su, pu = carry
                for j in range(L):
                    st = lax.while_loop(
                        hit_cond,
                        functools.partial(hit_body, ev=evs[j]),
                        (ms[j], su, pu))
                    su, pu = st[1], st[2]
                return (su, pu)

            return lax.fori_loop(0, nvmax, avreg, carry)

        def wpair(wp, carry):
            for par in range(2):
                w = wp * 2 + par
                buf, sem = wins[par], wsems[par]

                def step(carry=carry, w=w, buf=buf, sem=sem):
                    drain(w, buf, sem)
                    carry = walk(w, buf, carry)

                    @pl.when(w + 2 < nwin)
                    def _():
                        issue(w + 2, buf, sem)

                    return carry

                carry = lax.cond(w < nwin, step, lambda c=carry: c)
            return carry

        init = (jnp.int32(0), trash)
        nwp = (nwin + 1) >> 1
        su, pu = lax.fori_loop(0, nwp, wpair, init)

        six[0, pl.ds(0, L)] = pu
        pltpu.async_copy(stg, uv_hbm.at[six.at[0]], sem_s).wait()

    return sc_kernel


def _tc_score(u_ref, v_ref, r_ref, o_ref):
    u = u_ref[:, :DIM]
    v = v_ref[:, :DIM]
    rh = r_ref[:, :DIM]
    rt = r_ref[:, DIM:]
    hn = jnp.sqrt(jnp.sum(u * u, axis=1, keepdims=True))
    tn = jnp.sqrt(jnp.sum(v * v, axis=1, keepdims=True))
    un = u / jnp.maximum(hn, 1e-12)
    vn = v / jnp.maximum(tn, 1e-12)
    o_ref[...] = -jnp.sum(jnp.abs(un * rh - vn * rt), axis=1, keepdims=True)


@functools.lru_cache(maxsize=None)
def _build_tc(batch):
    blk = 512
    nblk = batch // blk
    return pl.pallas_call(
        _tc_score,
        grid=(nblk,),
        in_specs=[
            pl.BlockSpec((blk, 2 * DIM), lambda i: (i, 0)),
            pl.BlockSpec((blk, 2 * DIM), lambda i, n=nblk: (i + n, 0)),
            pl.BlockSpec((blk, 2 * DIM), lambda i: (i, 0)),
        ],
        out_specs=pl.BlockSpec((blk, 1), lambda i: (i, 0)),
        out_shape=jax.ShapeDtypeStruct((batch, 1), jnp.float32),
    )


def kernel(h, r, t, entity_emb, relation_emb):
    batch = h.shape[0]
    n_entity = entity_emb.shape[0]
    tail_base = (n_entity // TILE) * TILE
    et = entity_emb.T
    tail = lax.slice(entity_emb, (tail_base, 0), (n_entity, DIM)).T
    tail = jnp.pad(tail, ((0, 0), (0, TILE - tail.shape[1])))
    uv, ro = _build_sc(batch, n_entity)(h, r, t, et, relation_emb, tail)
    return _build_tc(batch)(uv, uv, ro)


# speedup vs baseline: 1.7054x; 1.0278x over previous
import functools

import jax
import jax.numpy as jnp
from jax import lax
from jax.experimental import pallas as pl
from jax.experimental.pallas import tpu as pltpu
from jax.experimental.pallas import tpu_sc as plsc

DIM = 64
L = 16
NC = 2
NS = 16
NW = NC * NS
TILE = 128
WTILES = 4
WIN = WTILES * TILE
RCHUNK = 128


def _cdiv(a, b):
    return (a + b - 1) // b


@functools.lru_cache(maxsize=None)
def _build_sc(batch, n_entity):
    bpw = batch // NW
    ntiles = _cdiv(n_entity, TILE)
    full_tiles = n_entity // TILE
    tail_base = full_tiles * TILE
    npieces = _cdiv(batch, 8192)
    rcap = 2 * (batch // L) + L
    mesh = plsc.VectorSubcoreMesh(core_axis_name="c", subcore_axis_name="s")

    out_types = (
        jax.ShapeDtypeStruct((2 * batch + NW, 2 * DIM), jnp.float32),
        jax.ShapeDtypeStruct((batch, 2 * DIM), jnp.float32),
    )

    @functools.partial(
        pl.kernel,
        out_type=out_types,
        mesh=mesh,
        compiler_params=pltpu.CompilerParams(needs_layout_passes=False),
        scratch_types=[
            pltpu.VMEM((DIM, WIN), jnp.float32),
            pltpu.VMEM((DIM, WIN), jnp.float32),
            pltpu.VMEM((L, rcap), jnp.int32),
            pltpu.VMEM((8192,), jnp.int32),
            pltpu.VMEM((L, 2 * DIM), jnp.float32),
            pltpu.VMEM((1, L), jnp.int32),
            pltpu.VMEM((RCHUNK, 2 * DIM), jnp.float32),
            pltpu.VMEM((bpw,), jnp.int32),
            pltpu.SemaphoreType.DMA,
            pltpu.SemaphoreType.DMA,
            pltpu.SemaphoreType.DMA,
            pltpu.SemaphoreType.DMA,
        ],
    )
    def sc_kernel(h_hbm, r_hbm, t_hbm, et_hbm, rel_hbm, tail_hbm,
                  uv_hbm, rout_hbm,
                  win0, win1, aren, piece, stg, six,
                  rel_v, ridx, sem_w0, sem_w1, sem_s, sem_r):
        wid = lax.axis_index("s") * NC + lax.axis_index("c")
        base = wid * bpw
        lane = lax.iota(jnp.int32, L)
        perms = [lane ^ k for k in (1, 2, 4, 8)]
        dnums = lax.GatherDimensionNumbers(
            offset_dims=(), collapsed_slice_dims=(0,), start_index_map=(0,))

        def lperm(v, p):
            return lax.gather(v, p[:, None], dnums, slice_sizes=(1,),
                              mode=lax.GatherScatterMode.PROMISE_IN_BOUNDS)

        pltpu.sync_copy(r_hbm.at[pl.ds(base, bpw)], ridx)
        for cc in range(bpw // RCHUNK):
            pltpu.async_copy(
                rel_hbm.at[ridx.at[pl.ds(cc * RCHUNK, RCHUNK)]], rel_v,
                sem_r).wait()
            pltpu.sync_copy(
                rel_v, rout_hbm.at[pl.ds(base + cc * RCHUNK, RCHUNK), :])

        t0 = (wid * ntiles) // NW
        t1 = ((wid + 1) * ntiles) // NW
        t1n = jnp.minimum(t1, jnp.int32(full_tiles))
        has_tail = jnp.where(t1 > t1n, jnp.int32(1), jnp.int32(0))
        nwin = (t1n - t0 + (WTILES - 1)) // WTILES + has_tail

        UNROLL = 4

        def scan_piece(p, cnt, src_hbm, tbl_bit):
            pltpu.sync_copy(src_hbm.at[pl.ds(p * 8192, 8192)], piece)

            def vstep(i, cnt):
                for s in range(UNROLL):
                    ii = i * UNROLL + s
                    v = piece[pl.ds(pl.multiple_of(ii * L, L), L)]
                    tl = v >> 7
                    m = (tl >= t0) & (tl < t1)
                    is_tl = tl >= jnp.int32(full_tiles)
                    wv = jnp.where(is_tl, nwin - 1, (tl - t0) >> 2)
                    tcv = jnp.minimum(t0 + ((tl - t0) >> 2) * WTILES,
                                      t1n - WTILES)
                    colv = jnp.where(is_tl, v - jnp.int32(tail_base),
                                     v - tcv * TILE)
                    pos = jnp.int32(p * 8192) + ii * L + lane
                    entry = (wv << 24) | (colv << 15) | tbl_bit | pos
                    entry = jnp.where(m, entry, jnp.int32(63 << 24))
                    dest = jnp.where(m, cnt, jnp.int32(rcap - L))
                    plsc.store_scatter(aren, [lane, dest], entry)
                    cnt = cnt + jnp.where(m, 1, 0)
                return cnt

            return lax.fori_loop(0, 8192 // (L * UNROLL), vstep, cnt)

        sentinel = jnp.broadcast_to(jnp.int32(63 << 24), (L,))

        def ainit(g, c):
            for j in range(L):
                aren[j, pl.ds(pl.multiple_of(g * L, L), L)] = sentinel
            return c

        lax.fori_loop(0, rcap // L, ainit, 0)

        cnt = jnp.zeros((L,), jnp.int32)
        for p in range(npieces):
            cnt = scan_piece(p, cnt, h_hbm, jnp.int32(0))
        for p in range(npieces):
            cnt = scan_piece(p, cnt, t_hbm, jnp.int32(1 << 14))

        trash = jnp.broadcast_to(jnp.int32(2 * batch), (L,)) + wid
        wins = (win0, win1)
        wsems = (sem_w0, sem_w1)

        def issue(w, buf, sem):
            is_tail = (w == nwin - 1) & (has_tail == 1)

            @pl.when(jnp.logical_not(is_tail))
            def _():
                tc = jnp.minimum(t0 + w * WTILES, t1n - WTILES)
                cb = pl.multiple_of(tc * TILE, TILE)
                pltpu.async_copy(et_hbm.at[:, pl.ds(cb, WIN)], buf, sem)

            @pl.when(is_tail)
            def _():
                pltpu.async_copy(tail_hbm, buf.at[:, pl.ds(0, TILE)], sem)

        def drain(w, buf, sem):
            is_tail = (w == nwin - 1) & (has_tail == 1)

            @pl.when(jnp.logical_not(is_tail))
            def _():
                pltpu.make_async_copy(
                    et_hbm.at[:, pl.ds(0, WIN)], buf, sem).wait()

            @pl.when(is_tail)
            def _():
                pltpu.make_async_copy(
                    tail_hbm, buf.at[:, pl.ds(0, TILE)], sem).wait()

        issue(jnp.int32(0), win0, sem_w0)

        @pl.when(nwin > 1)
        def _():
            issue(jnp.int32(1), win1, sem_w1)

        cmax = cnt
        for pp in perms:
            cmax = jnp.maximum(cmax, lperm(cmax, pp))
        nvmax = (cmax[0] + (L - 1)) >> 4

        def walk(w, buf, carry):

            def avreg(g, carry):
                goff = pl.multiple_of(g * L, L)
                evs = [aren[j, pl.ds(goff, L)] for j in range(L)]
                ms = [(ev >> 24) == w for ev in evs]

                def hit_cond(st):
                    return jnp.any(st[0])

                def hit_body(st, ev=None):
                    m, su, pu = st
                    mn = jnp.where(m, lane, jnp.int32(L))
                    for pp in perms:
                        mn = jnp.minimum(mn, lperm(mn, pp))
                    mn = jnp.minimum(mn, jnp.int32(L - 1))
                    p_v = lperm(ev, mn)
                    ent = p_v[0]
                    b = ent & jnp.int32((1 << 15) - 1)
                    col = jnp.broadcast_to((ent >> 15), (L,)) & 511
                    for k in range(DIM // L):
                        stg[su, pl.ds(k * L, L)] = plsc.load_gather(
                            buf, [lane + jnp.int32(k * L), col])
                    pu = jnp.where(lane == su, jnp.broadcast_to(b, (L,)),
                                   pu)

                    @pl.when(su == L - 1)
                    def _(pu=pu):
                        six[0, pl.ds(0, L)] = pu
                        pltpu.async_copy(
                            stg, uv_hbm.at[six.at[0]], sem_s).wait()

                    su2 = (su + 1) & (L - 1)
                    pu2 = jnp.where(su == L - 1, trash, pu)
                    m2 = m & (lane != mn)
                    return (m2, su2, pu2)

                su, pu = carry
                for j in range(L):
                    st = lax.while_loop(
                        hit_cond,
                        functools.partial(hit_body, ev=evs[j]),
                        (ms[j], su, pu))
                    su, pu = st[1], st[2]
                return (su, pu)

            return lax.fori_loop(0, nvmax, avreg, carry)

        def wpair(wp, carry):
            for par in range(2):
                w = wp * 2 + par
                buf, sem = wins[par], wsems[par]

                def step(carry=carry, w=w, buf=buf, sem=sem):
                    drain(w, buf, sem)
                    carry = walk(w, buf, carry)

                    @pl.when(w + 2 < nwin)
                    def _():
                        issue(w + 2, buf, sem)

                    return carry

                carry = lax.cond(w < nwin, step, lambda c=carry: c)
            return carry

        init = (jnp.int32(0), trash)
        nwp = (nwin + 1) >> 1
        su, pu = lax.fori_loop(0, nwp, wpair, init)

        six[0, pl.ds(0, L)] = pu
        pltpu.async_copy(stg, uv_hbm.at[six.at[0]], sem_s).wait()

    return sc_kernel


def _tc_score(u_ref, v_ref, r_ref, o_ref):
    u = u_ref[:, :DIM]
    v = v_ref[:, :DIM]
    rh = r_ref[:, :DIM]
    rt = r_ref[:, DIM:]
    hn = jnp.sqrt(jnp.sum(u * u, axis=1, keepdims=True))
    tn = jnp.sqrt(jnp.sum(v * v, axis=1, keepdims=True))
    un = u / jnp.maximum(hn, 1e-12)
    vn = v / jnp.maximum(tn, 1e-12)
    o_ref[...] = -jnp.sum(jnp.abs(un * rh - vn * rt), axis=1, keepdims=True)


@functools.lru_cache(maxsize=None)
def _build_tc(batch):
    blk = 512
    nblk = batch // blk
    return pl.pallas_call(
        _tc_score,
        grid=(nblk,),
        in_specs=[
            pl.BlockSpec((blk, 2 * DIM), lambda i: (i, 0)),
            pl.BlockSpec((blk, 2 * DIM), lambda i, n=nblk: (i + n, 0)),
            pl.BlockSpec((blk, 2 * DIM), lambda i: (i, 0)),
        ],
        out_specs=pl.BlockSpec((blk, 1), lambda i: (i, 0)),
        out_shape=jax.ShapeDtypeStruct((batch, 1), jnp.float32),
    )


def kernel(h, r, t, entity_emb, relation_emb):
    batch = h.shape[0]
    n_entity = entity_emb.shape[0]
    tail_base = (n_entity // TILE) * TILE
    et = entity_emb.T
    tail = lax.slice(entity_emb, (tail_base, 0), (n_entity, DIM)).T
    tail = jnp.pad(tail, ((0, 0), (0, TILE - tail.shape[1])))
    uv, ro = _build_sc(batch, n_entity)(h, r, t, et, relation_emb, tail)
    return _build_tc(batch)(uv, uv, ro)


# speedup vs baseline: 1.8433x; 1.0809x over previous
import functools

import jax
import jax.numpy as jnp
from jax import lax
from jax.experimental import pallas as pl
from jax.experimental.pallas import tpu as pltpu
from jax.experimental.pallas import tpu_sc as plsc

DIM = 64
L = 16
NC = 2
NS = 16
NW = NC * NS
TILE = 128
WTILES = 4
WIN = WTILES * TILE
RCHUNK = 128


def _cdiv(a, b):
    return (a + b - 1) // b


@functools.lru_cache(maxsize=None)
def _build_sc(batch, n_entity):
    bpw = batch // NW
    ntiles = _cdiv(n_entity, TILE)
    full_tiles = n_entity // TILE
    tail_base = full_tiles * TILE
    npieces = _cdiv(batch, 8192)
    rcap = 2 * (batch // L) + L
    mesh = plsc.VectorSubcoreMesh(core_axis_name="c", subcore_axis_name="s")

    out_types = (
        jax.ShapeDtypeStruct((2 * batch + NW, 2 * DIM), jnp.float32),
        jax.ShapeDtypeStruct((batch, 2 * DIM), jnp.float32),
    )

    @functools.partial(
        pl.kernel,
        out_type=out_types,
        mesh=mesh,
        compiler_params=pltpu.CompilerParams(needs_layout_passes=False),
        scratch_types=[
            pltpu.VMEM((DIM, WIN), jnp.float32),
            pltpu.VMEM((DIM, WIN), jnp.float32),
            pltpu.VMEM((L, rcap), jnp.int32),
            pltpu.VMEM((8192,), jnp.int32),
            pltpu.VMEM((L, 2 * DIM), jnp.float32),
            pltpu.VMEM((1, L), jnp.int32),
            pltpu.VMEM((RCHUNK, 2 * DIM), jnp.float32),
            pltpu.VMEM((bpw,), jnp.int32),
            pltpu.SemaphoreType.DMA,
            pltpu.SemaphoreType.DMA,
            pltpu.SemaphoreType.DMA,
            pltpu.SemaphoreType.DMA,
        ],
    )
    def sc_kernel(h_hbm, r_hbm, t_hbm, et_hbm, rel_hbm, tail_hbm,
                  uv_hbm, rout_hbm,
                  win0, win1, aren, piece, stg, six,
                  rel_v, ridx, sem_w0, sem_w1, sem_s, sem_r):
        wid = lax.axis_index("s") * NC + lax.axis_index("c")
        base = wid * bpw
        lane = lax.iota(jnp.int32, L)
        perms = [lane ^ k for k in (1, 2, 4, 8)]
        dnums = lax.GatherDimensionNumbers(
            offset_dims=(), collapsed_slice_dims=(0,), start_index_map=(0,))

        def lperm(v, p):
            return lax.gather(v, p[:, None], dnums, slice_sizes=(1,),
                              mode=lax.GatherScatterMode.PROMISE_IN_BOUNDS)

        pltpu.sync_copy(r_hbm.at[pl.ds(base, bpw)], ridx)
        for cc in range(bpw // RCHUNK):
            pltpu.async_copy(
                rel_hbm.at[ridx.at[pl.ds(cc * RCHUNK, RCHUNK)]], rel_v,
                sem_r).wait()
            pltpu.sync_copy(
                rel_v, rout_hbm.at[pl.ds(base + cc * RCHUNK, RCHUNK), :])

        t0 = (wid * ntiles) // NW
        t1 = ((wid + 1) * ntiles) // NW
        t1n = jnp.minimum(t1, jnp.int32(full_tiles))
        has_tail = jnp.where(t1 > t1n, jnp.int32(1), jnp.int32(0))
        nwin = (t1n - t0 + (WTILES - 1)) // WTILES + has_tail

        UNROLL = 4

        def scan_piece(p, cnt, src_hbm, tbl_bit):
            pltpu.sync_copy(src_hbm.at[pl.ds(p * 8192, 8192)], piece)

            def vstep(i, cnt):
                for s in range(UNROLL):
                    ii = i * UNROLL + s
                    v = piece[pl.ds(pl.multiple_of(ii * L, L), L)]
                    tl = v >> 7
                    m = (tl >= t0) & (tl < t1)
                    is_tl = tl >= jnp.int32(full_tiles)
                    wv = jnp.where(is_tl, nwin - 1, (tl - t0) >> 2)
                    tcv = jnp.minimum(t0 + ((tl - t0) >> 2) * WTILES,
                                      t1n - WTILES)
                    colv = jnp.where(is_tl, v - jnp.int32(tail_base),
                                     v - tcv * TILE)
                    pos = jnp.int32(p * 8192) + ii * L + lane
                    entry = (wv << 24) | (colv << 15) | tbl_bit | pos
                    entry = jnp.where(m, entry, jnp.int32(63 << 24))
                    dest = jnp.where(m, cnt, jnp.int32(rcap - L))
                    plsc.store_scatter(aren, [lane, dest], entry)
                    cnt = cnt + jnp.where(m, 1, 0)
                return cnt

            return lax.fori_loop(0, 8192 // (L * UNROLL), vstep, cnt)

        sentinel = jnp.broadcast_to(jnp.int32(63 << 24), (L,))

        def ainit(g, c):
            for j in range(L):
                aren[j, pl.ds(pl.multiple_of(g * L, L), L)] = sentinel
            return c

        lax.fori_loop(0, rcap // L, ainit, 0)

        cnt = jnp.zeros((L,), jnp.int32)
        for p in range(npieces):
            cnt = scan_piece(p, cnt, h_hbm, jnp.int32(0))
        for p in range(npieces):
            cnt = scan_piece(p, cnt, t_hbm, jnp.int32(1 << 14))

        trash = jnp.broadcast_to(jnp.int32(2 * batch), (L,)) + wid
        wins = (win0, win1)
        wsems = (sem_w0, sem_w1)

        def issue(w, buf, sem):
            is_tail = (w == nwin - 1) & (has_tail == 1)

            @pl.when(jnp.logical_not(is_tail))
            def _():
                tc = jnp.minimum(t0 + w * WTILES, t1n - WTILES)
                cb = pl.multiple_of(tc * TILE, TILE)
                pltpu.async_copy(et_hbm.at[:, pl.ds(cb, WIN)], buf, sem)

            @pl.when(is_tail)
            def _():
                pltpu.async_copy(tail_hbm, buf.at[:, pl.ds(0, TILE)], sem)

        def drain(w, buf, sem):
            is_tail = (w == nwin - 1) & (has_tail == 1)

            @pl.when(jnp.logical_not(is_tail))
            def _():
                pltpu.make_async_copy(
                    et_hbm.at[:, pl.ds(0, WIN)], buf, sem).wait()

            @pl.when(is_tail)
            def _():
                pltpu.make_async_copy(
                    tail_hbm, buf.at[:, pl.ds(0, TILE)], sem).wait()

        issue(jnp.int32(0), win0, sem_w0)

        @pl.when(nwin > 1)
        def _():
            issue(jnp.int32(1), win1, sem_w1)

        cmax = cnt
        for pp in perms:
            cmax = jnp.maximum(cmax, lperm(cmax, pp))
        nvmax = (cmax[0] + (L - 1)) >> 4

        def walk(w, buf, carry):

            def avreg(g, carry):
                goff = pl.multiple_of(g * L, L)
                evs = [aren[j, pl.ds(goff, L)] for j in range(L)]
                ms = [(ev >> 24) == w for ev in evs]

                def hit_cond(st):
                    return jnp.any(st[0])

                def hit_body(st, ev=None):
                    m, su, pu = st
                    mn = jnp.where(m, lane, jnp.int32(L))
                    for pp in perms:
                        mn = jnp.minimum(mn, lperm(mn, pp))
                    mn = jnp.minimum(mn, jnp.int32(L - 1))
                    p_v = lperm(ev, mn)
                    ent = p_v[0]
                    b = ent & jnp.int32((1 << 15) - 1)
                    col = jnp.broadcast_to((ent >> 15), (L,)) & 511
                    for k in range(DIM // L):
                        stg[su, pl.ds(k * L, L)] = plsc.load_gather(
                            buf, [lane + jnp.int32(k * L), col])
                    pu = jnp.where(lane == su, jnp.broadcast_to(b, (L,)),
                                   pu)

                    @pl.when(su == L - 1)
                    def _(pu=pu):
                        six[0, pl.ds(0, L)] = pu
                        pltpu.async_copy(
                            stg, uv_hbm.at[six.at[0]], sem_s).wait()

                    su2 = (su + 1) & (L - 1)
                    pu2 = jnp.where(su == L - 1, trash, pu)
                    m2 = m & (lane != mn)
                    return (m2, su2, pu2)

                su, pu = carry
                for q in range(4):
                    grp = ms[4 * q]
                    for j in range(4 * q + 1, 4 * q + 4):
                        grp = grp | ms[j]

                    def qbody(su=su, pu=pu, q=q):
                        for j in range(4 * q, 4 * q + 4):
                            st = lax.while_loop(
                                hit_cond,
                                functools.partial(hit_body, ev=evs[j]),
                                (ms[j], su, pu))
                            su, pu = st[1], st[2]
                        return (su, pu)

                    su, pu = lax.cond(jnp.any(grp), qbody,
                                      lambda su=su, pu=pu: (su, pu))
                return (su, pu)

            return lax.fori_loop(0, nvmax, avreg, carry)

        def wpair(wp, carry):
            for par in range(2):
                w = wp * 2 + par
                buf, sem = wins[par], wsems[par]

                def step(carry=carry, w=w, buf=buf, sem=sem):
                    drain(w, buf, sem)
                    carry = walk(w, buf, carry)

                    @pl.when(w + 2 < nwin)
                    def _():
                        issue(w + 2, buf, sem)

                    return carry

                carry = lax.cond(w < nwin, step, lambda c=carry: c)
            return carry

        init = (jnp.int32(0), trash)
        nwp = (nwin + 1) >> 1
        su, pu = lax.fori_loop(0, nwp, wpair, init)

        six[0, pl.ds(0, L)] = pu
        pltpu.async_copy(stg, uv_hbm.at[six.at[0]], sem_s).wait()

    return sc_kernel


def _tc_score(u_ref, v_ref, r_ref, o_ref):
    u = u_ref[:, :DIM]
    v = v_ref[:, :DIM]
    rh = r_ref[:, :DIM]
    rt = r_ref[:, DIM:]
    hn = jnp.sqrt(jnp.sum(u * u, axis=1, keepdims=True))
    tn = jnp.sqrt(jnp.sum(v * v, axis=1, keepdims=True))
    un = u / jnp.maximum(hn, 1e-12)
    vn = v / jnp.maximum(tn, 1e-12)
    o_ref[...] = -jnp.sum(jnp.abs(un * rh - vn * rt), axis=1, keepdims=True)


@functools.lru_cache(maxsize=None)
def _build_tc(batch):
    blk = 512
    nblk = batch // blk
    return pl.pallas_call(
        _tc_score,
        grid=(nblk,),
        in_specs=[
            pl.BlockSpec((blk, 2 * DIM), lambda i: (i, 0)),
            pl.BlockSpec((blk, 2 * DIM), lambda i, n=nblk: (i + n, 0)),
            pl.BlockSpec((blk, 2 * DIM), lambda i: (i, 0)),
        ],
        out_specs=pl.BlockSpec((blk, 1), lambda i: (i, 0)),
        out_shape=jax.ShapeDtypeStruct((batch, 1), jnp.float32),
    )


def kernel(h, r, t, entity_emb, relation_emb):
    batch = h.shape[0]
    n_entity = entity_emb.shape[0]
    tail_base = (n_entity // TILE) * TILE
    et = entity_emb.T
    tail = lax.slice(entity_emb, (tail_base, 0), (n_entity, DIM)).T
    tail = jnp.pad(tail, ((0, 0), (0, TILE - tail.shape[1])))
    uv, ro = _build_sc(batch, n_entity)(h, r, t, et, relation_emb, tail)
    return _build_tc(batch)(uv, uv, ro)
